# baseline (speedup 1.0000x reference)
import jax
import jax.numpy as jnp
from jax import lax
from jax.experimental import pallas as pl
from jax.experimental.pallas import tpu as pltpu
from jax.experimental.pallas import tpu_sc as plsc

_N = 50000
_E = 800000
_H = 128
_OUT = 4

_BN = 512
_BE = 1024
_N_PAD = 50176
_E_PAD = 819200


def _silu(v):
    return v * jax.nn.sigmoid(v)


def _ln(v, g, b, eps=1e-5):
    mu = jnp.mean(v, axis=-1, keepdims=True)
    var = jnp.mean((v - mu) ** 2, axis=-1, keepdims=True)
    return (v - mu) / jnp.sqrt(var + eps) * g + b


def _full_spec(shape):
    return pl.BlockSpec(shape, lambda i: (0,) * len(shape))


def _row_spec(rows, cols):
    return pl.BlockSpec((rows, cols), lambda i: (i, 0))


def _enc_body(x_ref, w1, b1, w2, b2, g, b, o_ref):
    t = _silu(jnp.dot(x_ref[...], w1[...], preferred_element_type=jnp.float32)
              + b1[...])
    m = jnp.dot(t, w2[...], preferred_element_type=jnp.float32) + b2[...]
    o_ref[...] = _ln(m, g[...], b[...])


def _encode(x, p, rows, blk):
    din = x.shape[1]
    return pl.pallas_call(
        _enc_body,
        grid=(rows // blk,),
        in_specs=[
            _row_spec(blk, din),
            _full_spec((din, _H)), _full_spec((1, _H)),
            _full_spec((_H, _H)), _full_spec((1, _H)),
            _full_spec((1, _H)), _full_spec((1, _H)),
        ],
        out_specs=_row_spec(blk, _H),
        out_shape=jax.ShapeDtypeStruct((rows, _H), jnp.float32),
    )(x, p["l1"]["W"], p["l1"]["b"].reshape(1, _H),
      p["l2"]["W"], p["l2"]["b"].reshape(1, _H),
      p["ln"]["g"].reshape(1, _H), p["ln"]["b"].reshape(1, _H))


def _ab_body(h_ref, wa, wb, a_ref, b_ref):
    h = h_ref[...]
    a_ref[...] = jnp.dot(h, wa[...], preferred_element_type=jnp.float32)
    b_ref[...] = jnp.dot(h, wb[...], preferred_element_type=jnp.float32)


def _ab(h, wa, wb):
    return pl.pallas_call(
        _ab_body,
        grid=(_N_PAD // _BN,),
        in_specs=[_row_spec(_BN, _H), _full_spec((_H, _H)), _full_spec((_H, _H))],
        out_specs=(_row_spec(_BN, _H), _row_spec(_BN, _H)),
        out_shape=(jax.ShapeDtypeStruct((_N_PAD, _H), jnp.float32),
                   jax.ShapeDtypeStruct((_N_PAD, _H), jnp.float32)),
    )(h, wa, wb)


def _edge_body(pre_ref, e_ref, w1c, b1, w2, b2, g, b, o_ref):
    t = pre_ref[...] + jnp.dot(e_ref[...], w1c[...],
                               preferred_element_type=jnp.float32) + b1[...]
    t = _silu(t)
    m = jnp.dot(t, w2[...], preferred_element_type=jnp.float32) + b2[...]
    o_ref[...] = _ln(m, g[...], b[...])


def _edge_mlp(pre, e, w1c, b1, w2, b2, g, b):
    return pl.pallas_call(
        _edge_body,
        grid=(_E_PAD // _BE,),
        in_specs=[
            _row_spec(_BE, _H), _row_spec(_BE, _H),
            _full_spec((_H, _H)), _full_spec((1, _H)),
            _full_spec((_H, _H)), _full_spec((1, _H)),
            _full_spec((1, _H)), _full_spec((1, _H)),
        ],
        out_specs=_row_spec(_BE, _H),
        out_shape=jax.ShapeDtypeStruct((_E_PAD, _H), jnp.float32),
    )(pre, e, w1c, b1.reshape(1, _H), w2, b2.reshape(1, _H),
      g.reshape(1, _H), b.reshape(1, _H))


def _p2_spec(q):
    return pl.BlockSpec((1, _BN, _H), lambda i, q=q: (q, i, 0))


def _node_body(h_ref, a0_ref, a1_ref, c0_ref, c1_ref, w1h, w1a, b1, w2, b2,
               g, b, o_ref):
    cnt = c0_ref[0][:, 0:1] + c1_ref[0][:, 0:1]
    cnt = jnp.maximum(cnt, 1.0)
    aggm = (a0_ref[0] + a1_ref[0]) / cnt
    h = h_ref[...]
    t = (jnp.dot(h, w1h[...], preferred_element_type=jnp.float32)
         + jnp.dot(aggm, w1a[...], preferred_element_type=jnp.float32)
         + b1[...])
    t = _silu(t)
    m = jnp.dot(t, w2[...], preferred_element_type=jnp.float32) + b2[...]
    o_ref[...] = h + _ln(m, g[...], b[...])


def _node_mlp(h, aggp, cntp, w1h, w1a, b1, w2, b2, g, b):
    return pl.pallas_call(
        _node_body,
        grid=(_N_PAD // _BN,),
        in_specs=[
            _row_spec(_BN, _H), _p2_spec(0), _p2_spec(1),
            _p2_spec(0), _p2_spec(1),
            _full_spec((_H, _H)), _full_spec((_H, _H)), _full_spec((1, _H)),
            _full_spec((_H, _H)), _full_spec((1, _H)),
            _full_spec((1, _H)), _full_spec((1, _H)),
        ],
        out_specs=_row_spec(_BN, _H),
        out_shape=jax.ShapeDtypeStruct((_N_PAD, _H), jnp.float32),
    )(h, aggp, aggp, cntp, cntp, w1h, w1a, b1.reshape(1, _H), w2,
      b2.reshape(1, _H), g.reshape(1, _H), b.reshape(1, _H))


def _dec_body(h_ref, w1, b1, w2, b2, w3, b3, o_ref):
    o = _silu(jnp.dot(h_ref[...], w1[...], preferred_element_type=jnp.float32)
              + b1[...])
    o = _silu(jnp.dot(o, w2[...], preferred_element_type=jnp.float32) + b2[...])
    o_ref[...] = jnp.dot(o, w3[...], preferred_element_type=jnp.float32) + b3[...]


def _decode(h, d):
    h2 = _H // 2
    return pl.pallas_call(
        _dec_body,
        grid=(_N_PAD // _BN,),
        in_specs=[
            _row_spec(_BN, _H),
            _full_spec((_H, _H)), _full_spec((1, _H)),
            _full_spec((_H, h2)), _full_spec((1, h2)),
            _full_spec((h2, _OUT)), _full_spec((1, _OUT)),
        ],
        out_specs=_row_spec(_BN, _OUT),
        out_shape=jax.ShapeDtypeStruct((_N_PAD, _OUT), jnp.float32),
    )(h, d["l1"]["W"], d["l1"]["b"].reshape(1, _H),
      d["l2"]["W"], d["l2"]["b"].reshape(1, h2),
      d["l3"]["W"], d["l3"]["b"].reshape(1, _OUT))


_NW = 32
_GC = 64
_GNC = _E_PAD // _NW // _GC
_HNC = _GNC // 2

_SC_MESH = plsc.VectorSubcoreMesh(core_axis_name="c", subcore_axis_name="s")


def _gather_kernel_body(a_hbm, b_hbm, dst_hbm, src_hbm, out_hbm,
                        dsti, srci,
                        ba0, bb0, ba1, bb1, ba2, bb2, ba3, bb3,
                        gsem0, gsem1, gsem2, gsem3,
                        osem0, osem1, osem2, osem3):
    c = lax.axis_index("c")
    s = lax.axis_index("s")
    wid = s * 2 + c
    rowbase = wid * _GNC
    slots = ((ba0, bb0, gsem0, osem0),
             (ba1, bb1, gsem1, osem1),
             (ba2, bb2, gsem2, osem2),
             (ba3, bb3, gsem3, osem3))

    def fire(j, slot):
        ba, bb, gsem, _ = slots[slot]
        pltpu.async_copy(a_hbm.at[dsti.at[j, 0]], ba, gsem)
        pltpu.async_copy(b_hbm.at[srci.at[j, 0]], bb, gsem)

    def waitg(slot):
        ba, bb, gsem, _ = slots[slot]
        pltpu.make_async_copy(a_hbm.at[dsti.at[0, 0]], ba, gsem).wait()
        pltpu.make_async_copy(b_hbm.at[srci.at[0, 0]], bb, gsem).wait()

    def add(slot):
        ba, bb = slots[slot][0], slots[slot][1]

        def body(r, _):
            for q in range(8):
                sl = pl.ds(q * 16, 16)
                ba[r, sl] = ba[r, sl] + bb[r, sl]
            return 0

        lax.fori_loop(0, _GC, body, 0)


    for half in (0, 1):
        pltpu.sync_copy(dst_hbm.at[pl.ds(rowbase + half * _HNC, _HNC)], dsti)
        pltpu.sync_copy(src_hbm.at[pl.ds(rowbase + half * _HNC, _HNC)], srci)
        ebase = (rowbase + half * _HNC) * _GC

        def firew(j, slot, ebase=ebase):
            ba, osem = slots[slot][0], slots[slot][3]
            pltpu.async_copy(ba, out_hbm.at[pl.ds(ebase + j * _GC, _GC)], osem)

        def waitw(slot):
            ba, osem = slots[slot][0], slots[slot][3]
            pltpu.make_async_copy(ba, out_hbm.at[pl.ds(0, _GC)], osem).wait()

        for t in range(4):
            fire(t, t)

        def step(jj, _):
            j0 = jj * 4
            for t in range(4):
                waitg(t)
                add(t)
                firew(j0 + t, t)
            for t in range(4):
                waitw(t)
                fire(j0 + 4 + t, t)
            return 0

        lax.fori_loop(0, _HNC // 4 - 1, step, 0)
        j0 = _HNC - 4
        for t in range(4):
            waitg(t)
            add(t)
            firew(j0 + t, t)
        for t in range(4):
            waitw(t)


def _gather_pre(a, b, dst3d, src3d):
    return pl.kernel(
        _gather_kernel_body,
        out_type=jax.ShapeDtypeStruct((_E_PAD, _H), jnp.float32),
        mesh=_SC_MESH,
        scratch_types=[
            pltpu.VMEM((_HNC, 1, _GC), jnp.int32),
            pltpu.VMEM((_HNC, 1, _GC), jnp.int32),
            pltpu.VMEM((_GC, _H), jnp.float32),
            pltpu.VMEM((_GC, _H), jnp.float32),
            pltpu.VMEM((_GC, _H), jnp.float32),
            pltpu.VMEM((_GC, _H), jnp.float32),
            pltpu.VMEM((_GC, _H), jnp.float32),
            pltpu.VMEM((_GC, _H), jnp.float32),
            pltpu.VMEM((_GC, _H), jnp.float32),
            pltpu.VMEM((_GC, _H), jnp.float32),
            pltpu.SemaphoreType.DMA,
            pltpu.SemaphoreType.DMA,
            pltpu.SemaphoreType.DMA,
            pltpu.SemaphoreType.DMA,
            pltpu.SemaphoreType.DMA,
            pltpu.SemaphoreType.DMA,
            pltpu.SemaphoreType.DMA,
            pltpu.SemaphoreType.DMA,
        ],
    )(a, b, dst3d, src3d)


def kernel(x, edge_index, edge_attr, params):
    src = edge_index[0]
    dst = edge_index[1]
    pe = _E_PAD - _E
    pad_node = _N_PAD - 1
    dst_p = jnp.concatenate([dst, jnp.full((pe,), pad_node, jnp.int32)])
    src_p = jnp.concatenate([src, jnp.full((pe,), pad_node, jnp.int32)])
    dst3d = dst_p.reshape(_E_PAD // _GC, 1, _GC)
    src3d = src_p.reshape(_E_PAD // _GC, 1, _GC)
    ea_p = jnp.pad(edge_attr, ((0, pe), (0, 0)))
    x_p = jnp.pad(x, ((0, _N_PAD - _N), (0, 0)))

    h = _encode(x_p, params["node_enc"], _N_PAD, _BN)
    e = _encode(ea_p, params["edge_enc"], _E_PAD, _BE)

    c = jax.ops.segment_sum(jnp.ones((_E_PAD,), jnp.float32), dst_p,
                            num_segments=_N_PAD)
    cntp = jnp.stack([jnp.broadcast_to(c[:, None], (_N_PAD, _H)),
                      jnp.zeros((_N_PAD, _H), jnp.float32)])

    for layer in params["mp"]:
        w1 = layer["edge_mlp"]["l1"]["W"]
        w1a, w1b, w1c = w1[:_H], w1[_H:2 * _H], w1[2 * _H:]
        a, bt = _ab(h, w1a, w1b)
        pre = _gather_pre(a, bt, dst3d, src3d)
        msg = _edge_mlp(pre, e, w1c,
                        layer["edge_mlp"]["l1"]["b"],
                        layer["edge_mlp"]["l2"]["W"],
                        layer["edge_mlp"]["l2"]["b"],
                        layer["edge_mlp"]["ln"]["g"],
                        layer["edge_mlp"]["ln"]["b"])
        srt = jax.ops.segment_sum(msg, dst_p, num_segments=_N_PAD)
        aggp = jnp.stack([srt, jnp.zeros((_N_PAD, _H), jnp.float32)])
        wn1 = layer["node_mlp"]["l1"]["W"]
        h = _node_mlp(h, aggp, cntp,
                      wn1[:_H], wn1[_H:],
                      layer["node_mlp"]["l1"]["b"],
                      layer["node_mlp"]["l2"]["W"],
                      layer["node_mlp"]["l2"]["b"],
                      layer["node_mlp"]["ln"]["g"],
                      layer["node_mlp"]["ln"]["b"])

    out = _decode(h, params["decoder"])
    return out[:_N]


# speedup vs baseline: 1.0155x; 1.0155x over previous
import jax
import jax.numpy as jnp
from jax import lax
from jax.experimental import pallas as pl
from jax.experimental.pallas import tpu as pltpu
from jax.experimental.pallas import tpu_sc as plsc

_N = 50000
_E = 800000
_H = 128
_OUT = 4

_BN = 512
_BE = 1024
_N_PAD = 50176
_E_PAD = 819200


def _silu(v):
    return v * jax.nn.sigmoid(v)


def _ln(v, g, b, eps=1e-5):
    mu = jnp.mean(v, axis=-1, keepdims=True)
    var = jnp.mean((v - mu) ** 2, axis=-1, keepdims=True)
    return (v - mu) / jnp.sqrt(var + eps) * g + b


def _full_spec(shape):
    return pl.BlockSpec(shape, lambda i: (0,) * len(shape))


def _row_spec(rows, cols):
    return pl.BlockSpec((rows, cols), lambda i: (i, 0))


def _enc_body(x_ref, w1, b1, w2, b2, g, b, o_ref):
    t = _silu(jnp.dot(x_ref[...], w1[...], preferred_element_type=jnp.float32)
              + b1[...])
    m = jnp.dot(t, w2[...], preferred_element_type=jnp.float32) + b2[...]
    o_ref[...] = _ln(m, g[...], b[...])


def _encode(x, p, rows, blk):
    din = x.shape[1]
    return pl.pallas_call(
        _enc_body,
        grid=(rows // blk,),
        in_specs=[
            _row_spec(blk, din),
            _full_spec((din, _H)), _full_spec((1, _H)),
            _full_spec((_H, _H)), _full_spec((1, _H)),
            _full_spec((1, _H)), _full_spec((1, _H)),
        ],
        out_specs=_row_spec(blk, _H),
        out_shape=jax.ShapeDtypeStruct((rows, _H), jnp.float32),
    )(x, p["l1"]["W"], p["l1"]["b"].reshape(1, _H),
      p["l2"]["W"], p["l2"]["b"].reshape(1, _H),
      p["ln"]["g"].reshape(1, _H), p["ln"]["b"].reshape(1, _H))


def _ab_body(h_ref, wa, wb, a_ref, b_ref):
    h = h_ref[...]
    a_ref[...] = jnp.dot(h, wa[...], preferred_element_type=jnp.float32)
    b_ref[...] = jnp.dot(h, wb[...], preferred_element_type=jnp.float32)


def _ab(h, wa, wb):
    return pl.pallas_call(
        _ab_body,
        grid=(_N_PAD // _BN,),
        in_specs=[_row_spec(_BN, _H), _full_spec((_H, _H)), _full_spec((_H, _H))],
        out_specs=(_row_spec(_BN, _H), _row_spec(_BN, _H)),
        out_shape=(jax.ShapeDtypeStruct((_N_PAD, _H), jnp.float32),
                   jax.ShapeDtypeStruct((_N_PAD, _H), jnp.float32)),
    )(h, wa, wb)


def _edge_body(pre_ref, e_ref, w1c, b1, w2, b2, g, b, o_ref):
    t = pre_ref[...] + jnp.dot(e_ref[...], w1c[...],
                               preferred_element_type=jnp.float32) + b1[...]
    t = _silu(t)
    m = jnp.dot(t, w2[...], preferred_element_type=jnp.float32) + b2[...]
    o_ref[...] = _ln(m, g[...], b[...])


def _edge_mlp(pre, e, w1c, b1, w2, b2, g, b):
    return pl.pallas_call(
        _edge_body,
        grid=(_E_PAD // _BE,),
        in_specs=[
            _row_spec(_BE, _H), _row_spec(_BE, _H),
            _full_spec((_H, _H)), _full_spec((1, _H)),
            _full_spec((_H, _H)), _full_spec((1, _H)),
            _full_spec((1, _H)), _full_spec((1, _H)),
        ],
        out_specs=_row_spec(_BE, _H),
        out_shape=jax.ShapeDtypeStruct((_E_PAD, _H), jnp.float32),
    )(pre, e, w1c, b1.reshape(1, _H), w2, b2.reshape(1, _H),
      g.reshape(1, _H), b.reshape(1, _H))


def _p2_spec(q):
    return pl.BlockSpec((1, _BN, _H), lambda i, q=q: (q, i, 0))


def _node_body(h_ref, a0_ref, a1_ref, c0_ref, c1_ref, w1h, w1a, b1, w2, b2,
               g, b, o_ref):
    cnt = c0_ref[0][:, 0:1] + c1_ref[0][:, 0:1]
    cnt = jnp.maximum(cnt, 1.0)
    aggm = (a0_ref[0] + a1_ref[0]) / cnt
    h = h_ref[...]
    t = (jnp.dot(h, w1h[...], preferred_element_type=jnp.float32)
         + jnp.dot(aggm, w1a[...], preferred_element_type=jnp.float32)
         + b1[...])
    t = _silu(t)
    m = jnp.dot(t, w2[...], preferred_element_type=jnp.float32) + b2[...]
    o_ref[...] = h + _ln(m, g[...], b[...])


def _node_mlp(h, aggp, cntp, w1h, w1a, b1, w2, b2, g, b):
    return pl.pallas_call(
        _node_body,
        grid=(_N_PAD // _BN,),
        in_specs=[
            _row_spec(_BN, _H), _p2_spec(0), _p2_spec(1),
            _p2_spec(0), _p2_spec(1),
            _full_spec((_H, _H)), _full_spec((_H, _H)), _full_spec((1, _H)),
            _full_spec((_H, _H)), _full_spec((1, _H)),
            _full_spec((1, _H)), _full_spec((1, _H)),
        ],
        out_specs=_row_spec(_BN, _H),
        out_shape=jax.ShapeDtypeStruct((_N_PAD, _H), jnp.float32),
    )(h, aggp, aggp, cntp, cntp, w1h, w1a, b1.reshape(1, _H), w2,
      b2.reshape(1, _H), g.reshape(1, _H), b.reshape(1, _H))


def _dec_body(h_ref, w1, b1, w2, b2, w3, b3, o_ref):
    o = _silu(jnp.dot(h_ref[...], w1[...], preferred_element_type=jnp.float32)
              + b1[...])
    o = _silu(jnp.dot(o, w2[...], preferred_element_type=jnp.float32) + b2[...])
    o_ref[...] = jnp.dot(o, w3[...], preferred_element_type=jnp.float32) + b3[...]


def _decode(h, d):
    h2 = _H // 2
    return pl.pallas_call(
        _dec_body,
        grid=(_N_PAD // _BN,),
        in_specs=[
            _row_spec(_BN, _H),
            _full_spec((_H, _H)), _full_spec((1, _H)),
            _full_spec((_H, h2)), _full_spec((1, h2)),
            _full_spec((h2, _OUT)), _full_spec((1, _OUT)),
        ],
        out_specs=_row_spec(_BN, _OUT),
        out_shape=jax.ShapeDtypeStruct((_N_PAD, _OUT), jnp.float32),
    )(h, d["l1"]["W"], d["l1"]["b"].reshape(1, _H),
      d["l2"]["W"], d["l2"]["b"].reshape(1, h2),
      d["l3"]["W"], d["l3"]["b"].reshape(1, _OUT))


_NW = 32
_GC = 128
_GNC = _E_PAD // _NW // _GC

_SC_MESH = plsc.VectorSubcoreMesh(core_axis_name="c", subcore_axis_name="s")


def _gather_kernel_body(a_hbm, b_hbm, dst_hbm, src_hbm, out_hbm,
                        dsti, srci, ba0, bb0, ba1, bb1,
                        gsem0, gsem1, osem0, osem1):
    c = lax.axis_index("c")
    s = lax.axis_index("s")
    wid = s * 2 + c
    rowbase = wid * _GNC
    ebase = wid * (_GNC * _GC)
    pltpu.sync_copy(dst_hbm.at[pl.ds(rowbase, _GNC)], dsti)
    pltpu.sync_copy(src_hbm.at[pl.ds(rowbase, _GNC)], srci)

    slots = ((dsti, srci, ba0, bb0, gsem0, osem0),
             (dsti, srci, ba1, bb1, gsem1, osem1))

    def fire(j, slot):
        di, si, ba, bb, gsem, _ = slots[slot]
        pltpu.async_copy(a_hbm.at[di.at[j, 0]], ba, gsem)
        pltpu.async_copy(b_hbm.at[si.at[j, 0]], bb, gsem)

    def waitg(slot):
        di, si, ba, bb, gsem, _ = slots[slot]
        pltpu.make_async_copy(a_hbm.at[di.at[0, 0]], ba, gsem).wait()
        pltpu.make_async_copy(b_hbm.at[si.at[0, 0]], bb, gsem).wait()

    def add(slot):
        ba, bb = slots[slot][2], slots[slot][3]

        def body(r, _):
            for q in range(8):
                sl = pl.ds(q * 16, 16)
                ba[r, sl] = ba[r, sl] + bb[r, sl]
            return 0

        lax.fori_loop(0, _GC, body, 0)

    def firew(j, slot):
        ba, osem = slots[slot][2], slots[slot][5]
        pltpu.async_copy(ba, out_hbm.at[pl.ds(ebase + j * _GC, _GC)], osem)

    def waitw(slot):
        ba, osem = slots[slot][2], slots[slot][5]
        pltpu.make_async_copy(ba, out_hbm.at[pl.ds(ebase, _GC)], osem).wait()

    fire(0, 0)
    fire(1, 1)

    def step(jj, _):
        j0 = jj * 2
        waitg(0)
        add(0)
        firew(j0, 0)
        waitg(1)
        add(1)
        firew(j0 + 1, 1)
        waitw(0)
        fire(j0 + 2, 0)
        waitw(1)
        fire(j0 + 3, 1)
        return 0

    lax.fori_loop(0, _GNC // 2 - 1, step, 0)
    j0 = _GNC - 2
    waitg(0)
    add(0)
    firew(j0, 0)
    waitg(1)
    add(1)
    firew(j0 + 1, 1)
    waitw(0)
    waitw(1)


def _gather_pre(a, b, dst3d, src3d):
    return pl.kernel(
        _gather_kernel_body,
        out_type=jax.ShapeDtypeStruct((_E_PAD, _H), jnp.float32),
        mesh=_SC_MESH,
        scratch_types=[
            pltpu.VMEM((_GNC, 1, _GC), jnp.int32),
            pltpu.VMEM((_GNC, 1, _GC), jnp.int32),
            pltpu.VMEM((_GC, _H), jnp.float32),
            pltpu.VMEM((_GC, _H), jnp.float32),
            pltpu.VMEM((_GC, _H), jnp.float32),
            pltpu.VMEM((_GC, _H), jnp.float32),
            pltpu.SemaphoreType.DMA,
            pltpu.SemaphoreType.DMA,
            pltpu.SemaphoreType.DMA,
            pltpu.SemaphoreType.DMA,
        ],
    )(a, b, dst3d, src3d)


def kernel(x, edge_index, edge_attr, params):
    src = edge_index[0]
    dst = edge_index[1]
    pe = _E_PAD - _E
    pad_node = _N_PAD - 1
    dst_p = jnp.concatenate([dst, jnp.full((pe,), pad_node, jnp.int32)])
    src_p = jnp.concatenate([src, jnp.full((pe,), pad_node, jnp.int32)])
    dst3d = dst_p.reshape(_E_PAD // _GC, 1, _GC)
    src3d = src_p.reshape(_E_PAD // _GC, 1, _GC)
    ea_p = jnp.pad(edge_attr, ((0, pe), (0, 0)))
    x_p = jnp.pad(x, ((0, _N_PAD - _N), (0, 0)))

    h = _encode(x_p, params["node_enc"], _N_PAD, _BN)
    e = _encode(ea_p, params["edge_enc"], _E_PAD, _BE)

    c = jax.ops.segment_sum(jnp.ones((_E_PAD,), jnp.float32), dst_p,
                            num_segments=_N_PAD)
    cntp = jnp.stack([jnp.broadcast_to(c[:, None], (_N_PAD, _H)),
                      jnp.zeros((_N_PAD, _H), jnp.float32)])

    for layer in params["mp"]:
        w1 = layer["edge_mlp"]["l1"]["W"]
        w1a, w1b, w1c = w1[:_H], w1[_H:2 * _H], w1[2 * _H:]
        a, bt = _ab(h, w1a, w1b)
        pre = _gather_pre(a, bt, dst3d, src3d)
        msg = _edge_mlp(pre, e, w1c,
                        layer["edge_mlp"]["l1"]["b"],
                        layer["edge_mlp"]["l2"]["W"],
                        layer["edge_mlp"]["l2"]["b"],
                        layer["edge_mlp"]["ln"]["g"],
                        layer["edge_mlp"]["ln"]["b"])
        srt = jax.ops.segment_sum(msg, dst_p, num_segments=_N_PAD)
        aggp = jnp.stack([srt, jnp.zeros((_N_PAD, _H), jnp.float32)])
        wn1 = layer["node_mlp"]["l1"]["W"]
        h = _node_mlp(h, aggp, cntp,
                      wn1[:_H], wn1[_H:],
                      layer["node_mlp"]["l1"]["b"],
                      layer["node_mlp"]["l2"]["W"],
                      layer["node_mlp"]["l2"]["b"],
                      layer["node_mlp"]["ln"]["g"],
                      layer["node_mlp"]["ln"]["b"])

    out = _decode(h, params["decoder"])
    return out[:_N]


# speedup vs baseline: 1.0339x; 1.0181x over previous
import jax
import jax.numpy as jnp
from jax import lax
from jax.experimental import pallas as pl
from jax.experimental.pallas import tpu as pltpu
from jax.experimental.pallas import tpu_sc as plsc

_N = 50000
_E = 800000
_H = 128
_OUT = 4

_BN = 512
_BE = 1024
_N_PAD = 50176
_E_PAD = 819200


def _silu(v):
    return v * jax.nn.sigmoid(v)


def _ln(v, g, b, eps=1e-5):
    mu = jnp.mean(v, axis=-1, keepdims=True)
    var = jnp.mean((v - mu) ** 2, axis=-1, keepdims=True)
    return (v - mu) / jnp.sqrt(var + eps) * g + b


def _full_spec(shape):
    return pl.BlockSpec(shape, lambda i: (0,) * len(shape))


def _row_spec(rows, cols):
    return pl.BlockSpec((rows, cols), lambda i: (i, 0))


def _enc_body(x_ref, w1, b1, w2, b2, g, b, o_ref):
    t = _silu(jnp.dot(x_ref[...], w1[...], preferred_element_type=jnp.float32)
              + b1[...])
    m = jnp.dot(t, w2[...], preferred_element_type=jnp.float32) + b2[...]
    o_ref[...] = _ln(m, g[...], b[...])


def _encode(x, p, rows, blk):
    din = x.shape[1]
    return pl.pallas_call(
        _enc_body,
        grid=(rows // blk,),
        in_specs=[
            _row_spec(blk, din),
            _full_spec((din, _H)), _full_spec((1, _H)),
            _full_spec((_H, _H)), _full_spec((1, _H)),
            _full_spec((1, _H)), _full_spec((1, _H)),
        ],
        out_specs=_row_spec(blk, _H),
        out_shape=jax.ShapeDtypeStruct((rows, _H), jnp.float32),
    )(x, p["l1"]["W"], p["l1"]["b"].reshape(1, _H),
      p["l2"]["W"], p["l2"]["b"].reshape(1, _H),
      p["ln"]["g"].reshape(1, _H), p["ln"]["b"].reshape(1, _H))


def _ab_body(h_ref, wa, wb, a_ref, b_ref):
    h = h_ref[...]
    a_ref[...] = jnp.dot(h, wa[...], preferred_element_type=jnp.float32)
    b_ref[...] = jnp.dot(h, wb[...], preferred_element_type=jnp.float32)


def _ab(h, wa, wb):
    return pl.pallas_call(
        _ab_body,
        grid=(_N_PAD // _BN,),
        in_specs=[_row_spec(_BN, _H), _full_spec((_H, _H)), _full_spec((_H, _H))],
        out_specs=(_row_spec(_BN, _H), _row_spec(_BN, _H)),
        out_shape=(jax.ShapeDtypeStruct((_N_PAD, _H), jnp.float32),
                   jax.ShapeDtypeStruct((_N_PAD, _H), jnp.float32)),
    )(h, wa, wb)


def _edge_body(pre_ref, e_ref, w1c, b1, w2, b2, g, b, o_ref):
    t = pre_ref[...] + jnp.dot(e_ref[...], w1c[...],
                               preferred_element_type=jnp.float32) + b1[...]
    t = _silu(t)
    m = jnp.dot(t, w2[...], preferred_element_type=jnp.float32) + b2[...]
    o_ref[...] = _ln(m, g[...], b[...])


def _edge_mlp(pre, e, w1c, b1, w2, b2, g, b):
    return pl.pallas_call(
        _edge_body,
        grid=(_E_PAD // _BE,),
        in_specs=[
            _row_spec(_BE, _H), _row_spec(_BE, _H),
            _full_spec((_H, _H)), _full_spec((1, _H)),
            _full_spec((_H, _H)), _full_spec((1, _H)),
            _full_spec((1, _H)), _full_spec((1, _H)),
        ],
        out_specs=_row_spec(_BE, _H),
        out_shape=jax.ShapeDtypeStruct((_E_PAD, _H), jnp.float32),
    )(pre, e, w1c, b1.reshape(1, _H), w2, b2.reshape(1, _H),
      g.reshape(1, _H), b.reshape(1, _H))


def _p2_spec(q):
    return pl.BlockSpec((1, _BN, _H), lambda i, q=q: (q, i, 0))


def _node_body(h_ref, a0_ref, a1_ref, c0_ref, c1_ref, w1h, w1a, b1, w2, b2,
               g, b, wan, wbn, o_ref, an_ref, bn_ref):
    cnt = c0_ref[0][:, 0:1] + c1_ref[0][:, 0:1]
    cnt = jnp.maximum(cnt, 1.0)
    aggm = (a0_ref[0] + a1_ref[0]) / cnt
    h = h_ref[...]
    t = (jnp.dot(h, w1h[...], preferred_element_type=jnp.float32)
         + jnp.dot(aggm, w1a[...], preferred_element_type=jnp.float32)
         + b1[...])
    t = _silu(t)
    m = jnp.dot(t, w2[...], preferred_element_type=jnp.float32) + b2[...]
    hn = h + _ln(m, g[...], b[...])
    o_ref[...] = hn
    an_ref[...] = jnp.dot(hn, wan[...], preferred_element_type=jnp.float32)
    bn_ref[...] = jnp.dot(hn, wbn[...], preferred_element_type=jnp.float32)


def _node_mlp(h, aggp, cntp, w1h, w1a, b1, w2, b2, g, b, wan, wbn):
    return pl.pallas_call(
        _node_body,
        grid=(_N_PAD // _BN,),
        in_specs=[
            _row_spec(_BN, _H), _p2_spec(0), _p2_spec(1),
            _p2_spec(0), _p2_spec(1),
            _full_spec((_H, _H)), _full_spec((_H, _H)), _full_spec((1, _H)),
            _full_spec((_H, _H)), _full_spec((1, _H)),
            _full_spec((1, _H)), _full_spec((1, _H)),
            _full_spec((_H, _H)), _full_spec((_H, _H)),
        ],
        out_specs=(_row_spec(_BN, _H), _row_spec(_BN, _H), _row_spec(_BN, _H)),
        out_shape=(jax.ShapeDtypeStruct((_N_PAD, _H), jnp.float32),
                   jax.ShapeDtypeStruct((_N_PAD, _H), jnp.float32),
                   jax.ShapeDtypeStruct((_N_PAD, _H), jnp.float32)),
    )(h, aggp, aggp, cntp, cntp, w1h, w1a, b1.reshape(1, _H), w2,
      b2.reshape(1, _H), g.reshape(1, _H), b.reshape(1, _H), wan, wbn)


def _dec_body(h_ref, w1, b1, w2, b2, w3, b3, o_ref):
    o = _silu(jnp.dot(h_ref[...], w1[...], preferred_element_type=jnp.float32)
              + b1[...])
    o = _silu(jnp.dot(o, w2[...], preferred_element_type=jnp.float32) + b2[...])
    o_ref[...] = jnp.dot(o, w3[...], preferred_element_type=jnp.float32) + b3[...]


def _decode(h, d):
    h2 = _H // 2
    return pl.pallas_call(
        _dec_body,
        grid=(_N_PAD // _BN,),
        in_specs=[
            _row_spec(_BN, _H),
            _full_spec((_H, _H)), _full_spec((1, _H)),
            _full_spec((_H, h2)), _full_spec((1, h2)),
            _full_spec((h2, _OUT)), _full_spec((1, _OUT)),
        ],
        out_specs=_row_spec(_BN, _OUT),
        out_shape=jax.ShapeDtypeStruct((_N_PAD, _OUT), jnp.float32),
    )(h, d["l1"]["W"], d["l1"]["b"].reshape(1, _H),
      d["l2"]["W"], d["l2"]["b"].reshape(1, h2),
      d["l3"]["W"], d["l3"]["b"].reshape(1, _OUT))


_NW = 32
_GC = 128
_GNC = _E_PAD // _NW // _GC

_SC_MESH = plsc.VectorSubcoreMesh(core_axis_name="c", subcore_axis_name="s")


def _gather_kernel_body(a_hbm, b_hbm, dst_hbm, src_hbm, out_hbm,
                        dsti, srci, ba0, bb0, ba1, bb1,
                        gsem0, gsem1, osem0, osem1):
    c = lax.axis_index("c")
    s = lax.axis_index("s")
    wid = s * 2 + c
    rowbase = wid * _GNC
    ebase = wid * (_GNC * _GC)
    pltpu.sync_copy(dst_hbm.at[pl.ds(rowbase, _GNC)], dsti)
    pltpu.sync_copy(src_hbm.at[pl.ds(rowbase, _GNC)], srci)

    slots = ((dsti, srci, ba0, bb0, gsem0, osem0),
             (dsti, srci, ba1, bb1, gsem1, osem1))

    def fire(j, slot):
        di, si, ba, bb, gsem, _ = slots[slot]
        pltpu.async_copy(a_hbm.at[di.at[j, 0]], ba, gsem)
        pltpu.async_copy(b_hbm.at[si.at[j, 0]], bb, gsem)

    def waitg(slot):
        di, si, ba, bb, gsem, _ = slots[slot]
        pltpu.make_async_copy(a_hbm.at[di.at[0, 0]], ba, gsem).wait()
        pltpu.make_async_copy(b_hbm.at[si.at[0, 0]], bb, gsem).wait()

    def add(slot):
        ba, bb = slots[slot][2], slots[slot][3]

        def body(r, _):
            for q in range(8):
                sl = pl.ds(q * 16, 16)
                ba[r, sl] = ba[r, sl] + bb[r, sl]
            return 0

        lax.fori_loop(0, _GC, body, 0)

    def firew(j, slot):
        ba, osem = slots[slot][2], slots[slot][5]
        pltpu.async_copy(ba, out_hbm.at[pl.ds(ebase + j * _GC, _GC)], osem)

    def waitw(slot):
        ba, osem = slots[slot][2], slots[slot][5]
        pltpu.make_async_copy(ba, out_hbm.at[pl.ds(ebase, _GC)], osem).wait()

    fire(0, 0)
    fire(1, 1)

    def step(jj, _):
        j0 = jj * 2
        waitg(0)
        add(0)
        firew(j0, 0)
        waitg(1)
        add(1)
        firew(j0 + 1, 1)
        waitw(0)
        fire(j0 + 2, 0)
        waitw(1)
        fire(j0 + 3, 1)
        return 0

    lax.fori_loop(0, _GNC // 2 - 1, step, 0)
    j0 = _GNC - 2
    waitg(0)
    add(0)
    firew(j0, 0)
    waitg(1)
    add(1)
    firew(j0 + 1, 1)
    waitw(0)
    waitw(1)


def _gather_pre(a, b, dst3d, src3d):
    return pl.kernel(
        _gather_kernel_body,
        out_type=jax.ShapeDtypeStruct((_E_PAD, _H), jnp.float32),
        mesh=_SC_MESH,
        scratch_types=[
            pltpu.VMEM((_GNC, 1, _GC), jnp.int32),
            pltpu.VMEM((_GNC, 1, _GC), jnp.int32),
            pltpu.VMEM((_GC, _H), jnp.float32),
            pltpu.VMEM((_GC, _H), jnp.float32),
            pltpu.VMEM((_GC, _H), jnp.float32),
            pltpu.VMEM((_GC, _H), jnp.float32),
            pltpu.SemaphoreType.DMA,
            pltpu.SemaphoreType.DMA,
            pltpu.SemaphoreType.DMA,
            pltpu.SemaphoreType.DMA,
        ],
    )(a, b, dst3d, src3d)


def kernel(x, edge_index, edge_attr, params):
    src = edge_index[0]
    dst = edge_index[1]
    pe = _E_PAD - _E
    pad_node = _N_PAD - 1
    dst_p = jnp.concatenate([dst, jnp.full((pe,), pad_node, jnp.int32)])
    src_p = jnp.concatenate([src, jnp.full((pe,), pad_node, jnp.int32)])
    dst3d = dst_p.reshape(_E_PAD // _GC, 1, _GC)
    src3d = src_p.reshape(_E_PAD // _GC, 1, _GC)
    ea_p = jnp.pad(edge_attr, ((0, pe), (0, 0)))
    x_p = jnp.pad(x, ((0, _N_PAD - _N), (0, 0)))

    h = _encode(x_p, params["node_enc"], _N_PAD, _BN)
    e = _encode(ea_p, params["edge_enc"], _E_PAD, _BE)

    c = jax.ops.segment_sum(jnp.ones((_E_PAD,), jnp.float32), dst_p,
                            num_segments=_N_PAD)
    cntp = jnp.stack([jnp.broadcast_to(c[:, None], (_N_PAD, _H)),
                      jnp.zeros((_N_PAD, _H), jnp.float32)])

    mp = params["mp"]
    splits = []
    for layer in mp:
        w1 = layer["edge_mlp"]["l1"]["W"]
        splits.append((w1[:_H], w1[_H:2 * _H], w1[2 * _H:]))

    a, bt = _ab(h, splits[0][0], splits[0][1])
    for i, layer in enumerate(mp):
        w1c = splits[i][2]
        pre = _gather_pre(a, bt, dst3d, src3d)
        msg = _edge_mlp(pre, e, w1c,
                        layer["edge_mlp"]["l1"]["b"],
                        layer["edge_mlp"]["l2"]["W"],
                        layer["edge_mlp"]["l2"]["b"],
                        layer["edge_mlp"]["ln"]["g"],
                        layer["edge_mlp"]["ln"]["b"])
        srt = jax.ops.segment_sum(msg, dst_p, num_segments=_N_PAD)
        aggp = jnp.stack([srt, jnp.zeros((_N_PAD, _H), jnp.float32)])
        wn1 = layer["node_mlp"]["l1"]["W"]
        nxt = splits[i + 1] if i + 1 < len(mp) else splits[i]
        h, a, bt = _node_mlp(h, aggp, cntp,
                             wn1[:_H], wn1[_H:],
                             layer["node_mlp"]["l1"]["b"],
                             layer["node_mlp"]["l2"]["W"],
                             layer["node_mlp"]["l2"]["b"],
                             layer["node_mlp"]["ln"]["g"],
                             layer["node_mlp"]["ln"]["b"],
                             nxt[0], nxt[1])

    out = _decode(h, params["decoder"])
    return out[:_N]


# speedup vs baseline: 1.0482x; 1.0139x over previous
import jax
import jax.numpy as jnp
from jax import lax
from jax.experimental import pallas as pl
from jax.experimental.pallas import tpu as pltpu
from jax.experimental.pallas import tpu_sc as plsc

_N = 50000
_E = 800000
_H = 128
_OUT = 4

_BN = 512
_BE = 1024
_N_PAD = 50176
_E_PAD = 819200


def _silu(v):
    return v * jax.nn.sigmoid(v)


def _ln(v, g, b, eps=1e-5):
    mu = jnp.mean(v, axis=-1, keepdims=True)
    var = jnp.mean((v - mu) ** 2, axis=-1, keepdims=True)
    return (v - mu) / jnp.sqrt(var + eps) * g + b


def _full_spec(shape):
    return pl.BlockSpec(shape, lambda i: (0,) * len(shape))


def _row_spec(rows, cols):
    return pl.BlockSpec((rows, cols), lambda i: (i, 0))


def _enc_body(x_ref, w1, b1, w2, b2, g, b, o_ref):
    t = _silu(jnp.dot(x_ref[...], w1[...], preferred_element_type=jnp.float32)
              + b1[...])
    m = jnp.dot(t, w2[...], preferred_element_type=jnp.float32) + b2[...]
    o_ref[...] = _ln(m, g[...], b[...])


def _encode(x, p, rows, blk):
    din = x.shape[1]
    return pl.pallas_call(
        _enc_body,
        grid=(rows // blk,),
        in_specs=[
            _row_spec(blk, din),
            _full_spec((din, _H)), _full_spec((1, _H)),
            _full_spec((_H, _H)), _full_spec((1, _H)),
            _full_spec((1, _H)), _full_spec((1, _H)),
        ],
        out_specs=_row_spec(blk, _H),
        out_shape=jax.ShapeDtypeStruct((rows, _H), jnp.float32),
    )(x, p["l1"]["W"], p["l1"]["b"].reshape(1, _H),
      p["l2"]["W"], p["l2"]["b"].reshape(1, _H),
      p["ln"]["g"].reshape(1, _H), p["ln"]["b"].reshape(1, _H))


def _ab_body(h_ref, wa, wb, a_ref, b_ref):
    h = h_ref[...]
    a_ref[...] = jnp.dot(h, wa[...], preferred_element_type=jnp.float32)
    b_ref[...] = jnp.dot(h, wb[...], preferred_element_type=jnp.float32)


def _ab(h, wa, wb):
    return pl.pallas_call(
        _ab_body,
        grid=(_N_PAD // _BN,),
        in_specs=[_row_spec(_BN, _H), _full_spec((_H, _H)), _full_spec((_H, _H))],
        out_specs=(_row_spec(_BN, _H), _row_spec(_BN, _H)),
        out_shape=(jax.ShapeDtypeStruct((_N_PAD, _H), jnp.float32),
                   jax.ShapeDtypeStruct((_N_PAD, _H), jnp.float32)),
    )(h, wa, wb)


def _edge_body(pre_ref, e_ref, w1c, b1, w2, b2, g, b, o_ref):
    t = pre_ref[...] + jnp.dot(e_ref[...], w1c[...],
                               preferred_element_type=jnp.float32) + b1[...]
    t = _silu(t)
    m = jnp.dot(t, w2[...], preferred_element_type=jnp.float32) + b2[...]
    o_ref[...] = _ln(m, g[...], b[...])


def _edge_mlp(pre, e, w1c, b1, w2, b2, g, b):
    return pl.pallas_call(
        _edge_body,
        grid=(_E_PAD // _BE,),
        in_specs=[
            _row_spec(_BE, _H), _row_spec(_BE, _H),
            _full_spec((_H, _H)), _full_spec((1, _H)),
            _full_spec((_H, _H)), _full_spec((1, _H)),
            _full_spec((1, _H)), _full_spec((1, _H)),
        ],
        out_specs=_row_spec(_BE, _H),
        out_shape=jax.ShapeDtypeStruct((_E_PAD, _H), jnp.float32),
    )(pre, e, w1c, b1.reshape(1, _H), w2, b2.reshape(1, _H),
      g.reshape(1, _H), b.reshape(1, _H))


def _node_body(h_ref, a0_ref, c0_ref, w1h, w1a, b1, w2, b2,
               g, b, wan, wbn, o_ref, an_ref, bn_ref):
    cnt = jnp.maximum(c0_ref[...][:, 0:1], 1.0)
    aggm = a0_ref[...] / cnt
    h = h_ref[...]
    t = (jnp.dot(h, w1h[...], preferred_element_type=jnp.float32)
         + jnp.dot(aggm, w1a[...], preferred_element_type=jnp.float32)
         + b1[...])
    t = _silu(t)
    m = jnp.dot(t, w2[...], preferred_element_type=jnp.float32) + b2[...]
    hn = h + _ln(m, g[...], b[...])
    o_ref[...] = hn
    an_ref[...] = jnp.dot(hn, wan[...], preferred_element_type=jnp.float32)
    bn_ref[...] = jnp.dot(hn, wbn[...], preferred_element_type=jnp.float32)


def _node_mlp(h, aggp, cntp, w1h, w1a, b1, w2, b2, g, b, wan, wbn):
    return pl.pallas_call(
        _node_body,
        grid=(_N_PAD // _BN,),
        in_specs=[
            _row_spec(_BN, _H), _row_spec(_BN, _H), _row_spec(_BN, _H),
            _full_spec((_H, _H)), _full_spec((_H, _H)), _full_spec((1, _H)),
            _full_spec((_H, _H)), _full_spec((1, _H)),
            _full_spec((1, _H)), _full_spec((1, _H)),
            _full_spec((_H, _H)), _full_spec((_H, _H)),
        ],
        out_specs=(_row_spec(_BN, _H), _row_spec(_BN, _H), _row_spec(_BN, _H)),
        out_shape=(jax.ShapeDtypeStruct((_N_PAD, _H), jnp.float32),
                   jax.ShapeDtypeStruct((_N_PAD, _H), jnp.float32),
                   jax.ShapeDtypeStruct((_N_PAD, _H), jnp.float32)),
    )(h, aggp, cntp, w1h, w1a, b1.reshape(1, _H), w2,
      b2.reshape(1, _H), g.reshape(1, _H), b.reshape(1, _H), wan, wbn)


def _dec_body(h_ref, w1, b1, w2, b2, w3, b3, o_ref):
    o = _silu(jnp.dot(h_ref[...], w1[...], preferred_element_type=jnp.float32)
              + b1[...])
    o = _silu(jnp.dot(o, w2[...], preferred_element_type=jnp.float32) + b2[...])
    o_ref[...] = jnp.dot(o, w3[...], preferred_element_type=jnp.float32) + b3[...]


def _decode(h, d):
    h2 = _H // 2
    return pl.pallas_call(
        _dec_body,
        grid=(_N_PAD // _BN,),
        in_specs=[
            _row_spec(_BN, _H),
            _full_spec((_H, _H)), _full_spec((1, _H)),
            _full_spec((_H, h2)), _full_spec((1, h2)),
            _full_spec((h2, _OUT)), _full_spec((1, _OUT)),
        ],
        out_specs=_row_spec(_BN, _OUT),
        out_shape=jax.ShapeDtypeStruct((_N_PAD, _OUT), jnp.float32),
    )(h, d["l1"]["W"], d["l1"]["b"].reshape(1, _H),
      d["l2"]["W"], d["l2"]["b"].reshape(1, h2),
      d["l3"]["W"], d["l3"]["b"].reshape(1, _OUT))


_NW = 32
_GC = 128
_GNC = _E_PAD // _NW // _GC

_SC_MESH = plsc.VectorSubcoreMesh(core_axis_name="c", subcore_axis_name="s")


def _gather_kernel_body(a_hbm, b_hbm, dst_hbm, src_hbm, out_hbm,
                        dsti, srci, ba0, bb0, ba1, bb1,
                        gsem0, gsem1, osem0, osem1):
    c = lax.axis_index("c")
    s = lax.axis_index("s")
    wid = s * 2 + c
    rowbase = wid * _GNC
    ebase = wid * (_GNC * _GC)
    pltpu.sync_copy(dst_hbm.at[pl.ds(rowbase, _GNC)], dsti)
    pltpu.sync_copy(src_hbm.at[pl.ds(rowbase, _GNC)], srci)

    slots = ((dsti, srci, ba0, bb0, gsem0, osem0),
             (dsti, srci, ba1, bb1, gsem1, osem1))

    def fire(j, slot):
        di, si, ba, bb, gsem, _ = slots[slot]
        pltpu.async_copy(a_hbm.at[di.at[j, 0]], ba, gsem)
        pltpu.async_copy(b_hbm.at[si.at[j, 0]], bb, gsem)

    def waitg(slot):
        di, si, ba, bb, gsem, _ = slots[slot]
        pltpu.make_async_copy(a_hbm.at[di.at[0, 0]], ba, gsem).wait()
        pltpu.make_async_copy(b_hbm.at[si.at[0, 0]], bb, gsem).wait()

    def add(slot):
        ba, bb = slots[slot][2], slots[slot][3]

        def body(r, _):
            for q in range(8):
                sl = pl.ds(q * 16, 16)
                ba[r, sl] = ba[r, sl] + bb[r, sl]
            return 0

        lax.fori_loop(0, _GC, body, 0)

    def firew(j, slot):
        ba, osem = slots[slot][2], slots[slot][5]
        pltpu.async_copy(ba, out_hbm.at[pl.ds(ebase + j * _GC, _GC)], osem)

    def waitw(slot):
        ba, osem = slots[slot][2], slots[slot][5]
        pltpu.make_async_copy(ba, out_hbm.at[pl.ds(ebase, _GC)], osem).wait()

    fire(0, 0)
    fire(1, 1)

    def step(jj, _):
        j0 = jj * 2
        waitg(0)
        add(0)
        firew(j0, 0)
        waitg(1)
        add(1)
        firew(j0 + 1, 1)
        waitw(0)
        fire(j0 + 2, 0)
        waitw(1)
        fire(j0 + 3, 1)
        return 0

    lax.fori_loop(0, _GNC // 2 - 1, step, 0)
    j0 = _GNC - 2
    waitg(0)
    add(0)
    firew(j0, 0)
    waitg(1)
    add(1)
    firew(j0 + 1, 1)
    waitw(0)
    waitw(1)


def _gather_pre(a, b, dst3d, src3d):
    return pl.kernel(
        _gather_kernel_body,
        out_type=jax.ShapeDtypeStruct((_E_PAD, _H), jnp.float32),
        mesh=_SC_MESH,
        scratch_types=[
            pltpu.VMEM((_GNC, 1, _GC), jnp.int32),
            pltpu.VMEM((_GNC, 1, _GC), jnp.int32),
            pltpu.VMEM((_GC, _H), jnp.float32),
            pltpu.VMEM((_GC, _H), jnp.float32),
            pltpu.VMEM((_GC, _H), jnp.float32),
            pltpu.VMEM((_GC, _H), jnp.float32),
            pltpu.SemaphoreType.DMA,
            pltpu.SemaphoreType.DMA,
            pltpu.SemaphoreType.DMA,
            pltpu.SemaphoreType.DMA,
        ],
    )(a, b, dst3d, src3d)


def kernel(x, edge_index, edge_attr, params):
    src = edge_index[0]
    dst = edge_index[1]
    pe = _E_PAD - _E
    pad_node = _N_PAD - 1
    dst_p = jnp.concatenate([dst, jnp.full((pe,), pad_node, jnp.int32)])
    src_p = jnp.concatenate([src, jnp.full((pe,), pad_node, jnp.int32)])
    dst3d = dst_p.reshape(_E_PAD // _GC, 1, _GC)
    src3d = src_p.reshape(_E_PAD // _GC, 1, _GC)
    ea_p = jnp.pad(edge_attr, ((0, pe), (0, 0)))
    x_p = jnp.pad(x, ((0, _N_PAD - _N), (0, 0)))

    h = _encode(x_p, params["node_enc"], _N_PAD, _BN)
    e = _encode(ea_p, params["edge_enc"], _E_PAD, _BE)

    c = jax.ops.segment_sum(jnp.ones((_E_PAD,), jnp.float32), dst_p,
                            num_segments=_N_PAD)
    cntp = jnp.broadcast_to(c[:, None], (_N_PAD, _H))

    mp = params["mp"]
    splits = []
    for layer in mp:
        w1 = layer["edge_mlp"]["l1"]["W"]
        splits.append((w1[:_H], w1[_H:2 * _H], w1[2 * _H:]))

    a, bt = _ab(h, splits[0][0], splits[0][1])
    for i, layer in enumerate(mp):
        w1c = splits[i][2]
        pre = _gather_pre(a, bt, dst3d, src3d)
        msg = _edge_mlp(pre, e, w1c,
                        layer["edge_mlp"]["l1"]["b"],
                        layer["edge_mlp"]["l2"]["W"],
                        layer["edge_mlp"]["l2"]["b"],
                        layer["edge_mlp"]["ln"]["g"],
                        layer["edge_mlp"]["ln"]["b"])
        aggp = jax.ops.segment_sum(msg, dst_p, num_segments=_N_PAD)
        wn1 = layer["node_mlp"]["l1"]["W"]
        nxt = splits[i + 1] if i + 1 < len(mp) else splits[i]
        h, a, bt = _node_mlp(h, aggp, cntp,
                             wn1[:_H], wn1[_H:],
                             layer["node_mlp"]["l1"]["b"],
                             layer["node_mlp"]["l2"]["W"],
                             layer["node_mlp"]["l2"]["b"],
                             layer["node_mlp"]["ln"]["g"],
                             layer["node_mlp"]["ln"]["b"],
                             nxt[0], nxt[1])

    out = _decode(h, params["decoder"])
    return out[:_N]
